# baseline (speedup 1.0000x reference)
import functools

import jax
import jax.numpy as jnp
from jax import lax
from jax.experimental import pallas as pl
from jax.experimental.pallas import tpu as pltpu
from jax.experimental.pallas import tpu_sc as plsc

_NC = 2
_NS = 16
_L = 16


@functools.lru_cache(maxsize=None)
def _build(V, D, B, S):
    NW = _NC * _NS
    CHP = S // NW
    BPW = B * CHP

    assert S % NW == 0 and CHP % _L == 0 and D % _L == 0 and CHP <= 128

    mesh = plsc.VectorSubcoreMesh(core_axis_name="c", subcore_axis_name="s")

    @functools.partial(
        pl.kernel,
        mesh=mesh,
        out_type=jax.ShapeDtypeStruct((B, S, D), jnp.float32),
        scratch_types=(
            [pltpu.VMEM((B, CHP), jnp.int32),
             pltpu.VMEM((CHP, D), jnp.float32),
             pltpu.VMEM((BPW, D), jnp.float32)]
            + [pltpu.SemaphoreType.DMA] * (B + 3)
        ),
    )
    def embed(idx_hbm, tok_hbm, pos_hbm, out_hbm, idx_v, pos_v, rows_v,
              *sems):
        gsems = sems[:B]
        p0sem, p1sem, isem = sems[B], sems[B + 1], sems[B + 2]
        wid = lax.axis_index("s") * _NC + lax.axis_index("c")
        p0 = wid * CHP

        icopies = [
            pltpu.async_copy(idx_hbm.at[bb, pl.ds(p0, CHP)], idx_v.at[bb],
                             isem)
            for bb in range(B)
        ]
        pre0 = pltpu.async_copy(pos_hbm.at[pl.ds(p0, CHP)],
                                rows_v.at[pl.ds(0, CHP)], p0sem)
        pre1 = pltpu.async_copy(pos_hbm.at[pl.ds(p0, CHP)], pos_v, p1sem)
        for ic in icopies:
            ic.wait()
        pre0.wait()
        gathers = [
            pltpu.async_copy(tok_hbm.at[idx_v.at[0]],
                             rows_v.at[pl.ds(0, CHP)], gsems[0], add=True)
        ]
        pre1.wait()

        def rep_for(boff):
            def rep(r, c):
                for u in range(4):
                    for k in range(D // _L):
                        sl = pl.ds(k * _L, _L)
                        rows_v[boff + 4 * r + u, sl] = pos_v[4 * r + u, sl]
                return c
            return rep

        for bb in range(1, B):
            lax.fori_loop(0, CHP // 4, rep_for(bb * CHP), 0)
            gathers.append(
                pltpu.async_copy(tok_hbm.at[idx_v.at[bb]],
                                 rows_v.at[pl.ds(bb * CHP, CHP)], gsems[bb],
                                 add=True))
        stores = []
        for bb in range(B):
            gathers[bb].wait()
            stores.append(
                pltpu.async_copy(rows_v.at[pl.ds(bb * CHP, CHP)],
                                 out_hbm.at[bb, pl.ds(p0, CHP)], isem))
        for st in stores:
            st.wait()

    return embed


def kernel(x, token_table, pos_table):
    B, S = x.shape
    V, D = token_table.shape
    return _build(V, D, B, S)(x.astype(jnp.int32), token_table, pos_table)


# speedup vs baseline: 1.0140x; 1.0140x over previous
import functools

import jax
import jax.numpy as jnp
from jax import lax
from jax.experimental import pallas as pl
from jax.experimental.pallas import tpu as pltpu
from jax.experimental.pallas import tpu_sc as plsc

_NC = 2
_NS = 16
_L = 16


@functools.lru_cache(maxsize=None)
def _build(V, D, B, S):
    NW = _NC * _NS
    CHP = S // NW
    BPW = B * CHP

    assert S % NW == 0 and CHP % _L == 0 and D % _L == 0 and CHP <= 128

    mesh = plsc.VectorSubcoreMesh(core_axis_name="c", subcore_axis_name="s")

    @functools.partial(
        pl.kernel,
        mesh=mesh,
        out_type=jax.ShapeDtypeStruct((B, S, D), jnp.float32),
        scratch_types=(
            [pltpu.VMEM((B, CHP), jnp.int32),
             pltpu.VMEM((CHP, D), jnp.float32),
             pltpu.VMEM((BPW, D), jnp.float32)]
            + [pltpu.SemaphoreType.DMA] * (B + 3)
        ),
    )
    def embed(idx_hbm, tok_hbm, pos_hbm, out_hbm, idx_v, pos_v, rows_v,
              *sems):
        gsems = sems[:B]
        p0sem, p1sem, isem = sems[B], sems[B + 1], sems[B + 2]
        wid = lax.axis_index("s") * _NC + lax.axis_index("c")
        p0 = wid * CHP

        icopies = [
            pltpu.async_copy(idx_hbm.at[bb, pl.ds(p0, CHP)], idx_v.at[bb],
                             isem)
            for bb in range(B)
        ]
        pre1 = pltpu.async_copy(pos_hbm.at[pl.ds(p0, CHP)], pos_v, p1sem)
        pre0 = pltpu.async_copy(pos_hbm.at[pl.ds(p0, CHP)],
                                rows_v.at[pl.ds(0, CHP)], p0sem)
        for ic in icopies:
            ic.wait()

        def rep_for(boff):
            def rep(r, c):
                for k in range(D // _L):
                    sl = pl.ds(k * _L, _L)
                    rows_v[boff + 2 * r, sl] = pos_v[2 * r, sl]
                    rows_v[boff + 2 * r + 1, sl] = pos_v[2 * r + 1, sl]
                return c
            return rep

        pre1.wait()
        lax.fori_loop(0, CHP // 2, rep_for(CHP), 0)
        pre0.wait()
        gathers = [
            pltpu.async_copy(tok_hbm.at[idx_v.at[0]],
                             rows_v.at[pl.ds(0, CHP)], gsems[0], add=True),
            pltpu.async_copy(tok_hbm.at[idx_v.at[1]],
                             rows_v.at[pl.ds(CHP, CHP)], gsems[1], add=True),
        ]
        for bb in range(2, B):
            lax.fori_loop(0, CHP // 2, rep_for(bb * CHP), 0)
            gathers.append(
                pltpu.async_copy(tok_hbm.at[idx_v.at[bb]],
                                 rows_v.at[pl.ds(bb * CHP, CHP)], gsems[bb],
                                 add=True))
        stores = []
        for bb in range(B):
            gathers[bb].wait()
            stores.append(
                pltpu.async_copy(rows_v.at[pl.ds(bb * CHP, CHP)],
                                 out_hbm.at[bb, pl.ds(p0, CHP)], isem))
        for st in stores:
            st.wait()

    return embed


def kernel(x, token_table, pos_table):
    B, S = x.shape
    V, D = token_table.shape
    return _build(V, D, B, S)(x.astype(jnp.int32), token_table, pos_table)
